# baseline (speedup 1.0000x reference)
import functools

import jax
import jax.numpy as jnp
from jax import lax
from jax.experimental import pallas as pl
from jax.experimental.pallas import tpu as pltpu
from jax.experimental.pallas import tpu_sc as plsc

B = 16384
L = 200
D = 64
NC = 2
NS = 16
NW = NC * NS
RPW = B // NW
BLK = 16
NBLK = RPW // BLK


def _sc_body(ids_hbm, table_hbm, out_hbm, ibuf, gbufA, gbufB, gbufC, gbufD,
             acc, semA, semB, semC, semD, isem):
    c = lax.axis_index("c")
    s = lax.axis_index("s")
    wid = s * NC + c
    base = wid * RPW

    def fire(h, t, dst, sem):
        pltpu.async_copy(table_hbm.at[ibuf.at[h, t]], dst, sem)

    def drain(dst, sem):
        pltpu.make_async_copy(table_hbm.at[ibuf.at[0, 0]], dst, sem).wait()

    def accum2(bx, by):
        @pl.loop(0, L)
        def _acc(p):
            for g in range(2):
                xi = plsc.bitcast(bx[p, pl.ds(g * 32, 32)], jnp.int32)
                yi = plsc.bitcast(by[p, pl.ds(g * 32, 32)], jnp.int32)
                lo = (lax.bitcast_convert_type(xi << 16, jnp.float32)
                      + lax.bitcast_convert_type(yi << 16, jnp.float32))
                hi = (lax.bitcast_convert_type(xi & -65536, jnp.float32)
                      + lax.bitcast_convert_type(yi & -65536, jnp.float32))
                sl = pl.ds(g * 32, 16)
                sh = pl.ds(g * 32 + 16, 16)
                acc[p, sl] = acc[p, sl] + lo
                acc[p, sh] = acc[p, sh] + hi

    pltpu.sync_copy(ids_hbm.at[pl.ds(base, BLK)], ibuf.at[0])

    @pl.loop(0, L)
    def _zero(p):
        for k2 in range(D // 16):
            acc[p, pl.ds(k2 * 16, 16)] = jnp.zeros((16,), jnp.float32)

    fire(0, 0, gbufA, semA)
    fire(0, 1, gbufB, semB)
    fire(0, 2, gbufC, semC)
    fire(0, 3, gbufD, semD)

    @pl.loop(0, NBLK)
    def _blk(kb):
        h = lax.rem(kb, 2)

        drain(gbufA, semA)
        drain(gbufB, semB)
        accum2(gbufA, gbufB)
        fire(h, 4, gbufA, semA)
        fire(h, 5, gbufB, semB)

        @pl.when(kb < NBLK - 1)
        def _pf():
            pltpu.async_copy(ids_hbm.at[pl.ds(base + (kb + 1) * BLK, BLK)],
                             ibuf.at[1 - h], isem)

        for j in range(1, 6):
            if j % 2 == 1:
                drain(gbufC, semC)
                drain(gbufD, semD)
                accum2(gbufC, gbufD)
                fire(h, 2 * j + 4, gbufC, semC)
                fire(h, 2 * j + 5, gbufD, semD)
            else:
                drain(gbufA, semA)
                drain(gbufB, semB)
                accum2(gbufA, gbufB)
                fire(h, 2 * j + 4, gbufA, semA)
                fire(h, 2 * j + 5, gbufB, semB)

        drain(gbufA, semA)
        drain(gbufB, semB)
        accum2(gbufA, gbufB)

        @pl.when(kb < NBLK - 1)
        def _nx0():
            pltpu.make_async_copy(
                ids_hbm.at[pl.ds(base, BLK)], ibuf.at[0], isem).wait()
            fire(1 - h, 0, gbufA, semA)
            fire(1 - h, 1, gbufB, semB)

        drain(gbufC, semC)
        drain(gbufD, semD)
        accum2(gbufC, gbufD)

        @pl.when(kb < NBLK - 1)
        def _nx1():
            fire(1 - h, 2, gbufC, semC)
            fire(1 - h, 3, gbufD, semD)

    pltpu.sync_copy(acc, out_hbm.at[wid])


_sc_embed = functools.partial(
    pl.kernel,
    out_type=jax.ShapeDtypeStruct((NW, L, D), jnp.float32),
    mesh=plsc.VectorSubcoreMesh(
        core_axis_name="c", subcore_axis_name="s",
        num_cores=NC, num_subcores=NS),
    compiler_params=pltpu.CompilerParams(use_tc_tiling_on_sc=False,
                                         needs_layout_passes=False),
    scratch_types=[
        pltpu.VMEM((2, BLK, L), jnp.int32),
        pltpu.VMEM((L, D), jnp.bfloat16),
        pltpu.VMEM((L, D), jnp.bfloat16),
        pltpu.VMEM((L, D), jnp.bfloat16),
        pltpu.VMEM((L, D), jnp.bfloat16),
        pltpu.VMEM((L, D), jnp.float32),
        pltpu.SemaphoreType.DMA,
        pltpu.SemaphoreType.DMA,
        pltpu.SemaphoreType.DMA,
        pltpu.SemaphoreType.DMA,
        pltpu.SemaphoreType.DMA,
    ],
)(_sc_body)


def _tc_combine(p_ref, o_ref):
    s = jnp.sum(p_ref[...], axis=0) * jnp.float32(1.0 / B)
    e = jax.lax.broadcasted_iota(jnp.int32, (L, D), 1)
    colmap = (e // 32) * 32 + (e % 2) * 16 + (e % 32) // 2
    o_ref[...] = jnp.take_along_axis(s, colmap, axis=1)


def kernel(token_ids, embedding_table):
    tbl_bf = embedding_table.astype(jnp.bfloat16)
    partial = _sc_embed(token_ids, tbl_bf)
    return pl.pallas_call(
        _tc_combine,
        out_shape=jax.ShapeDtypeStruct((L, D), jnp.float32),
    )(partial)


# speedup vs baseline: 1.5950x; 1.5950x over previous
import functools

import jax
import jax.numpy as jnp
from jax import lax
from jax.experimental import pallas as pl
from jax.experimental.pallas import tpu as pltpu
from jax.experimental.pallas import tpu_sc as plsc

B = 16384
L = 200
D = 64
NC = 2
NS = 16
NW = NC * NS
RPW = B // NW
BLK = 16
NBLK = RPW // BLK


def _sc_body(ids_hbm, table_hbm, out_hbm, ibuf, gbufA, gbufB, gbufC, gbufD,
             acc, semA, semB, semC, semD, isem):
    c = lax.axis_index("c")
    s = lax.axis_index("s")
    wid = s * NC + c
    base = wid * RPW

    def fire(h, t, dst, sem):
        pltpu.async_copy(table_hbm.at[ibuf.at[h, t]], dst, sem)

    def drain(dst, sem):
        pltpu.make_async_copy(table_hbm.at[ibuf.at[0, 0]], dst, sem).wait()

    def accum2(bx, by):
        @pl.loop(0, L)
        def _acc(p):
            for k2 in range(D // 16):
                sl = pl.ds(k2 * 16, 16)
                acc[p, sl] = acc[p, sl] + bx[p, sl] + by[p, sl]

    pltpu.sync_copy(ids_hbm.at[pl.ds(base, BLK)], ibuf.at[0])

    @pl.loop(0, L)
    def _zero(p):
        for k2 in range(D // 16):
            acc[p, pl.ds(k2 * 16, 16)] = jnp.zeros((16,), jnp.float32)

    fire(0, 0, gbufA, semA)
    fire(0, 1, gbufB, semB)
    fire(0, 2, gbufC, semC)
    fire(0, 3, gbufD, semD)

    @pl.loop(0, NBLK)
    def _blk(kb):
        h = lax.rem(kb, 2)

        drain(gbufA, semA)
        drain(gbufB, semB)
        accum2(gbufA, gbufB)
        fire(h, 4, gbufA, semA)
        fire(h, 5, gbufB, semB)

        @pl.when(kb < NBLK - 1)
        def _pf():
            pltpu.async_copy(ids_hbm.at[pl.ds(base + (kb + 1) * BLK, BLK)],
                             ibuf.at[1 - h], isem)

        for j in range(1, 6):
            if j % 2 == 1:
                drain(gbufC, semC)
                drain(gbufD, semD)
                accum2(gbufC, gbufD)
                fire(h, 2 * j + 4, gbufC, semC)
                fire(h, 2 * j + 5, gbufD, semD)
            else:
                drain(gbufA, semA)
                drain(gbufB, semB)
                accum2(gbufA, gbufB)
                fire(h, 2 * j + 4, gbufA, semA)
                fire(h, 2 * j + 5, gbufB, semB)

        drain(gbufA, semA)
        drain(gbufB, semB)
        accum2(gbufA, gbufB)

        @pl.when(kb < NBLK - 1)
        def _nx0():
            pltpu.make_async_copy(
                ids_hbm.at[pl.ds(base, BLK)], ibuf.at[0], isem).wait()
            fire(1 - h, 0, gbufA, semA)
            fire(1 - h, 1, gbufB, semB)

        drain(gbufC, semC)
        drain(gbufD, semD)
        accum2(gbufC, gbufD)

        @pl.when(kb < NBLK - 1)
        def _nx1():
            fire(1 - h, 2, gbufC, semC)
            fire(1 - h, 3, gbufD, semD)

    pltpu.sync_copy(acc, out_hbm.at[wid])


_sc_embed = functools.partial(
    pl.kernel,
    out_type=jax.ShapeDtypeStruct((NW, L, D), jnp.float32),
    mesh=plsc.VectorSubcoreMesh(
        core_axis_name="c", subcore_axis_name="s",
        num_cores=NC, num_subcores=NS),
    compiler_params=pltpu.CompilerParams(use_tc_tiling_on_sc=False),
    scratch_types=[
        pltpu.VMEM((2, BLK, L), jnp.int32),
        pltpu.VMEM((L, D), jnp.float32),
        pltpu.VMEM((L, D), jnp.float32),
        pltpu.VMEM((L, D), jnp.float32),
        pltpu.VMEM((L, D), jnp.float32),
        pltpu.VMEM((L, D), jnp.float32),
        pltpu.SemaphoreType.DMA,
        pltpu.SemaphoreType.DMA,
        pltpu.SemaphoreType.DMA,
        pltpu.SemaphoreType.DMA,
        pltpu.SemaphoreType.DMA,
    ],
)(_sc_body)


def _tc_combine(p_ref, o_ref):
    o_ref[...] = jnp.sum(p_ref[...], axis=0) * jnp.float32(1.0 / B)


def kernel(token_ids, embedding_table):
    partial = _sc_embed(token_ids, embedding_table)
    return pl.pallas_call(
        _tc_combine,
        out_shape=jax.ShapeDtypeStruct((L, D), jnp.float32),
    )(partial)
